# baseline (speedup 1.0000x reference)
import functools

import jax
import jax.numpy as jnp
from jax import lax
from jax.experimental import pallas as pl
from jax.experimental.pallas import tpu as pltpu
from jax.experimental.pallas import tpu_sc as plsc

NC = 2
NS = 16
NW = NC * NS

BATCH = 16384
FIELDS = 26
EMBED_DIM = 32
VOCAB = 1000000

PACK = 128 // EMBED_DIM
PROWS = VOCAB // PACK
NFC = VOCAB // 128
TAIL = VOCAB - NFC * 128
NBLK = NFC // 4
BLK_PER_W = (NBLK + NW - 1) // NW

CHUNK = 256
NCHW = (FIELDS * BATCH) // (CHUNK * NW)

_mesh = plsc.VectorSubcoreMesh(core_axis_name="c", subcore_axis_name="s")
_params = pltpu.CompilerParams(use_tc_tiling_on_sc=True,
                               needs_layout_passes=False)


def _wid():
    return lax.axis_index("s") * NC + lax.axis_index("c")


@functools.partial(
    pl.kernel,
    mesh=_mesh,
    out_type=jax.ShapeDtypeStruct((PROWS, 128), jnp.float32),
    scratch_types=[
        pltpu.VMEM((EMBED_DIM, 512), jnp.float32),
        pltpu.VMEM((EMBED_DIM, 512), jnp.float32),
        pltpu.VMEM((128, 128), jnp.float32),
        pltpu.VMEM((128, 128), jnp.float32),
    ]
    + [pltpu.SemaphoreType.DMA] * 4,
    compiler_params=_params,
)
def _pack_kernel(pt_hbm, tail_hbm, packed_hbm,
                 tbuf0, tbuf1, pbuf0, pbuf1,
                 isem0, isem1, osem0, osem1):
    wid = _wid()
    nt = jnp.where(wid < NBLK - (BLK_PER_W - 1) * NW, BLK_PER_W, BLK_PER_W - 1)
    iota16 = lax.iota(jnp.int32, 16)
    tbufs, pbufs = (tbuf0, tbuf1), (pbuf0, pbuf1)
    isem, osem = (isem0, isem1), (osem0, osem1)

    def in_slice(t):
        return pt_hbm.at[:, pl.ds((wid + NW * t) * 512, 512)]

    def out_slice(t):
        return packed_hbm.at[pl.ds((wid + NW * t) * 128, 128)]

    def transpose_block(tb, pb):
        def kk_group(kg, carry):
            for dk in range(8):
                kk = 8 * kg + dk
                col0 = 128 * lax.shift_right_logical(kk, 5) \
                    + 4 * lax.bitwise_and(kk, 31)
                for h in range(8):
                    rows = iota16 + 16 * (h % 2)
                    cols = jnp.full((16,), h // 2, jnp.int32) + col0
                    pb[kk, pl.ds(16 * h, 16)] = plsc.load_gather(
                        tb, [rows, cols])
            return carry

        lax.fori_loop(0, 16, kk_group, 0)

    def step(t, s):
        @pl.when(t < nt)
        def _():
            @pl.when(t >= 2)
            def _():
                pltpu.make_async_copy(pbufs[s], out_slice(t), osem[s]).wait()

            @pl.when(t + 1 < nt)
            def _():
                pltpu.async_copy(in_slice(t + 1), tbufs[1 - s], isem[1 - s])

            pltpu.make_async_copy(in_slice(t), tbufs[s], isem[s]).wait()
            transpose_block(tbufs[s], pbufs[s])
            pltpu.async_copy(pbufs[s], out_slice(t), osem[s])

    pltpu.async_copy(in_slice(0), tbuf0, isem0)

    def loop_body(u, carry):
        step(2 * u, 0)
        step(2 * u + 1, 1)
        return carry

    lax.fori_loop(0, (BLK_PER_W + 2) // 2, loop_body, 0)
    for s in range(2):
        pltpu.make_async_copy(pbufs[s], out_slice(0), osem[s]).wait()

    @pl.when(wid == 0)
    def _():
        pltpu.sync_copy(tail_hbm, pbuf0.at[pl.ds(0, TAIL // PACK)])
        pltpu.sync_copy(pbuf0.at[pl.ds(0, TAIL // PACK)],
                        packed_hbm.at[pl.ds(NFC * 32, TAIL // PACK)])


@functools.partial(
    pl.kernel,
    mesh=_mesh,
    out_type=jax.ShapeDtypeStruct((FIELDS, EMBED_DIM, BATCH), jnp.float32),
    scratch_types=[
        pltpu.VMEM((2, 128), jnp.int32),
        pltpu.VMEM((2, 128), jnp.int32),
        pltpu.VMEM((2, 128), jnp.int32),
        pltpu.VMEM((2, 128), jnp.int32),
        pltpu.VMEM((CHUNK, 128), jnp.float32),
        pltpu.VMEM((CHUNK, 128), jnp.float32),
        pltpu.VMEM((EMBED_DIM, CHUNK), jnp.float32),
        pltpu.VMEM((EMBED_DIM, CHUNK), jnp.float32),
    ]
    + [pltpu.SemaphoreType.DMA] * 6,
    compiler_params=_params,
)
def _gather_kernel(q_hbm, o_hbm, packed_hbm, out_hbm,
                   qbuf0, qbuf1, obuf0, obuf1, gbuf0, gbuf1, oblk0, oblk1,
                   ssem0, ssem1, gsem0, gsem1, osem0, osem1):
    wid = _wid()
    iota16 = lax.iota(jnp.int32, 16)
    qbufs, obufs = (qbuf0, qbuf1), (obuf0, obuf1)
    gbufs, oblks = (gbuf0, gbuf1), (oblk0, oblk1)
    ssem, gsem = (ssem0, ssem1), (gsem0, gsem1)
    osem = (osem0, osem1)

    def stage(t, s):
        pltpu.async_copy(q_hbm.at[wid, pl.ds(2 * t, 2)], qbufs[s], ssem[s])
        pltpu.async_copy(o_hbm.at[wid, pl.ds(2 * t, 2)], obufs[s], ssem[s])

    def fire(s):
        pltpu.async_copy(packed_hbm.at[qbufs[s].at[0]],
                         gbufs[s].at[pl.ds(0, 128)], gsem[s])
        pltpu.async_copy(packed_hbm.at[qbufs[s].at[1]],
                         gbufs[s].at[pl.ds(128, 128)], gsem[s])

    def wait_stage(s):
        pltpu.make_async_copy(q_hbm.at[wid, pl.ds(0, 2)], qbufs[s],
                              ssem[s]).wait()
        pltpu.make_async_copy(o_hbm.at[wid, pl.ds(0, 2)], obufs[s],
                              ssem[s]).wait()

    def out_slab(t):
        p = wid * NCHW + t
        f = p // (BATCH // CHUNK)
        bg = lax.rem(p, BATCH // CHUNK)
        return out_hbm.at[f, :, pl.ds(bg * CHUNK, CHUNK)]

    def step(t, s):
        @pl.when(t >= 2)
        def _():
            pltpu.make_async_copy(oblks[s], out_slab(t), osem[s]).wait()

        @pl.when(t + 1 < NCHW)
        def _():
            stage(t + 1, 1 - s)

        pltpu.make_async_copy(packed_hbm.at[pl.ds(0, CHUNK)], gbufs[s],
                              gsem[s]).wait()
        offs = [obufs[s][g // 8, pl.ds(16 * (g % 8), 16)] for g in range(16)]

        def e_group(eg, carry):
            for de in range(4):
                e = 4 * eg + de
                for g in range(16):
                    vals = plsc.load_gather(
                        gbufs[s], [iota16 + 16 * g, offs[g] + e])
                    oblks[s][e, pl.ds(16 * g, 16)] = vals
            return carry

        lax.fori_loop(0, 8, e_group, 0)
        pltpu.async_copy(oblks[s], out_slab(t), osem[s])

        @pl.when(t + 1 < NCHW)
        def _():
            wait_stage(1 - s)
            fire(1 - s)

    stage(0, 0)
    wait_stage(0)
    fire(0)

    def loop_body(u, carry):
        step(2 * u, 0)
        step(2 * u + 1, 1)
        return carry

    lax.fori_loop(0, NCHW // 2, loop_body, 0)
    for s in range(2):
        pltpu.make_async_copy(oblks[s], out_slab(0), osem[s]).wait()


def kernel(inputs, params):
    idxt = inputs.astype(jnp.int32).T.reshape(NW, NCHW * 2, 128)
    qarr = jnp.right_shift(idxt, 2)
    oarr = jnp.bitwise_and(idxt, 3) * EMBED_DIM
    tail = params[NFC * 128:].reshape(TAIL // PACK, 128)
    packed = _pack_kernel(params.T, tail)
    out_t = _gather_kernel(qarr, oarr, packed)
    return out_t.transpose(2, 0, 1)


# speedup vs baseline: 1.2570x; 1.2570x over previous
import functools

import jax
import jax.numpy as jnp
from jax import lax
from jax.experimental import pallas as pl
from jax.experimental.pallas import tpu as pltpu
from jax.experimental.pallas import tpu_sc as plsc

NC = 2
NS = 16
NW = NC * NS

BATCH = 16384
FIELDS = 26
EMBED_DIM = 32
VOCAB = 1000000

PACK = 128 // EMBED_DIM
PROWS = VOCAB // PACK
NFC = VOCAB // 128
TAIL = VOCAB - NFC * 128
NBLK = NFC // 4
BLK_PER_W = (NBLK + NW - 1) // NW

CHUNK = 256
NCHW = (FIELDS * BATCH) // (CHUNK * NW)

_mesh = plsc.VectorSubcoreMesh(core_axis_name="c", subcore_axis_name="s")
_params = pltpu.CompilerParams(use_tc_tiling_on_sc=True,
                               needs_layout_passes=False)


def _wid():
    return lax.axis_index("s") * NC + lax.axis_index("c")


@functools.partial(
    pl.kernel,
    mesh=_mesh,
    out_type=jax.ShapeDtypeStruct((PROWS, 128), jnp.float32),
    scratch_types=[
        pltpu.VMEM((EMBED_DIM, 512), jnp.float32),
        pltpu.VMEM((EMBED_DIM, 512), jnp.float32),
        pltpu.VMEM((128, 128), jnp.float32),
        pltpu.VMEM((128, 128), jnp.float32),
    ]
    + [pltpu.SemaphoreType.DMA] * 4,
    compiler_params=_params,
)
def _pack_kernel(pt_hbm, tail_hbm, packed_hbm,
                 tbuf0, tbuf1, pbuf0, pbuf1,
                 isem0, isem1, osem0, osem1):
    wid = _wid()
    nt = jnp.where(wid < NBLK - (BLK_PER_W - 1) * NW, BLK_PER_W, BLK_PER_W - 1)
    iota16 = lax.iota(jnp.int32, 16)
    tbufs, pbufs = (tbuf0, tbuf1), (pbuf0, pbuf1)
    isem, osem = (isem0, isem1), (osem0, osem1)

    def in_slice(t):
        return pt_hbm.at[:, pl.ds((wid + NW * t) * 512, 512)]

    def out_slice(t):
        return packed_hbm.at[pl.ds((wid + NW * t) * 128, 128)]

    def transpose_block(tb, pb):
        def kk_group(kg, carry):
            vals = []
            for dk in range(4):
                kk = 4 * kg + dk
                col0 = 128 * lax.shift_right_logical(kk, 5) \
                    + 4 * lax.bitwise_and(kk, 31)
                for h in range(8):
                    rows = iota16 + 16 * (h % 2)
                    cols = jnp.full((16,), h // 2, jnp.int32) + col0
                    vals.append(plsc.load_gather(tb, [rows, cols]))
            for dk in range(4):
                kk = 4 * kg + dk
                for h in range(8):
                    pb[kk, pl.ds(16 * h, 16)] = vals[8 * dk + h]
            return carry

        lax.fori_loop(0, 32, kk_group, 0)

    def step(t, s):
        @pl.when(t < nt)
        def _():
            @pl.when(t >= 2)
            def _():
                pltpu.make_async_copy(pbufs[s], out_slice(t), osem[s]).wait()

            @pl.when(t + 1 < nt)
            def _():
                pltpu.async_copy(in_slice(t + 1), tbufs[1 - s], isem[1 - s])

            pltpu.make_async_copy(in_slice(t), tbufs[s], isem[s]).wait()
            transpose_block(tbufs[s], pbufs[s])
            pltpu.async_copy(pbufs[s], out_slice(t), osem[s])

    pltpu.async_copy(in_slice(0), tbuf0, isem0)

    def loop_body(u, carry):
        step(2 * u, 0)
        step(2 * u + 1, 1)
        return carry

    lax.fori_loop(0, (BLK_PER_W + 2) // 2, loop_body, 0)
    for s in range(2):
        pltpu.make_async_copy(pbufs[s], out_slice(0), osem[s]).wait()

    @pl.when(wid == 0)
    def _():
        pltpu.sync_copy(tail_hbm, pbuf0.at[pl.ds(0, TAIL // PACK)])
        pltpu.sync_copy(pbuf0.at[pl.ds(0, TAIL // PACK)],
                        packed_hbm.at[pl.ds(NFC * 32, TAIL // PACK)])


@functools.partial(
    pl.kernel,
    mesh=_mesh,
    out_type=jax.ShapeDtypeStruct((FIELDS, EMBED_DIM, BATCH), jnp.float32),
    scratch_types=[
        pltpu.VMEM((2, 128), jnp.int32),
        pltpu.VMEM((2, 128), jnp.int32),
        pltpu.VMEM((2, 128), jnp.int32),
        pltpu.VMEM((2, 128), jnp.int32),
        pltpu.VMEM((CHUNK, 128), jnp.float32),
        pltpu.VMEM((CHUNK, 128), jnp.float32),
        pltpu.VMEM((EMBED_DIM, CHUNK), jnp.float32),
        pltpu.VMEM((EMBED_DIM, CHUNK), jnp.float32),
    ]
    + [pltpu.SemaphoreType.DMA] * 6,
    compiler_params=_params,
)
def _gather_kernel(q_hbm, o_hbm, packed_hbm, out_hbm,
                   qbuf0, qbuf1, obuf0, obuf1, gbuf0, gbuf1, oblk0, oblk1,
                   ssem0, ssem1, gsem0, gsem1, osem0, osem1):
    wid = _wid()
    iota16 = lax.iota(jnp.int32, 16)
    qbufs, obufs = (qbuf0, qbuf1), (obuf0, obuf1)
    gbufs, oblks = (gbuf0, gbuf1), (oblk0, oblk1)
    ssem, gsem = (ssem0, ssem1), (gsem0, gsem1)
    osem = (osem0, osem1)

    def stage(t, s):
        pltpu.async_copy(q_hbm.at[wid, pl.ds(2 * t, 2)], qbufs[s], ssem[s])
        pltpu.async_copy(o_hbm.at[wid, pl.ds(2 * t, 2)], obufs[s], ssem[s])

    def fire(s):
        pltpu.async_copy(packed_hbm.at[qbufs[s].at[0]],
                         gbufs[s].at[pl.ds(0, 128)], gsem[s])
        pltpu.async_copy(packed_hbm.at[qbufs[s].at[1]],
                         gbufs[s].at[pl.ds(128, 128)], gsem[s])

    def wait_stage(s):
        pltpu.make_async_copy(q_hbm.at[wid, pl.ds(0, 2)], qbufs[s],
                              ssem[s]).wait()
        pltpu.make_async_copy(o_hbm.at[wid, pl.ds(0, 2)], obufs[s],
                              ssem[s]).wait()

    def out_slab(t):
        p = wid * NCHW + t
        f = p // (BATCH // CHUNK)
        bg = lax.rem(p, BATCH // CHUNK)
        return out_hbm.at[f, :, pl.ds(bg * CHUNK, CHUNK)]

    def step(t, s):
        @pl.when(t >= 2)
        def _():
            pltpu.make_async_copy(oblks[s], out_slab(t), osem[s]).wait()

        @pl.when(t + 1 < NCHW)
        def _():
            stage(t + 1, 1 - s)

        pltpu.make_async_copy(packed_hbm.at[pl.ds(0, CHUNK)], gbufs[s],
                              gsem[s]).wait()
        offs = [obufs[s][g // 8, pl.ds(16 * (g % 8), 16)] for g in range(16)]

        def e_group(eg, carry):
            for de in range(4):
                e = 4 * eg + de
                for g in range(16):
                    vals = plsc.load_gather(
                        gbufs[s], [iota16 + 16 * g, offs[g] + e])
                    oblks[s][e, pl.ds(16 * g, 16)] = vals
            return carry

        lax.fori_loop(0, 8, e_group, 0)
        pltpu.async_copy(oblks[s], out_slab(t), osem[s])

        @pl.when(t + 1 < NCHW)
        def _():
            wait_stage(1 - s)
            fire(1 - s)

    stage(0, 0)
    wait_stage(0)
    fire(0)

    def loop_body(u, carry):
        step(2 * u, 0)
        step(2 * u + 1, 1)
        return carry

    lax.fori_loop(0, NCHW // 2, loop_body, 0)
    for s in range(2):
        pltpu.make_async_copy(oblks[s], out_slab(0), osem[s]).wait()


def kernel(inputs, params):
    idxt = inputs.astype(jnp.int32).T.reshape(NW, NCHW * 2, 128)
    qarr = jnp.right_shift(idxt, 2)
    oarr = jnp.bitwise_and(idxt, 3) * EMBED_DIM
    tail = params[NFC * 128:].reshape(TAIL // PACK, 128)
    packed = _pack_kernel(params.T, tail)
    out_t = _gather_kernel(qarr, oarr, packed)
    return out_t.transpose(2, 0, 1)


# speedup vs baseline: 1.3754x; 1.0942x over previous
import functools

import jax
import jax.numpy as jnp
from jax import lax
from jax.experimental import pallas as pl
from jax.experimental.pallas import tpu as pltpu
from jax.experimental.pallas import tpu_sc as plsc

NC = 2
NS = 16
NW = NC * NS

BATCH = 16384
FIELDS = 26
EMBED_DIM = 32
VOCAB = 1000000

PACK = 128 // EMBED_DIM
PROWS = VOCAB // PACK
NFC = VOCAB // 128
TAIL = VOCAB - NFC * 128
NBLK = NFC // 4
BLK_PER_W = (NBLK + NW - 1) // NW

CHUNK = 256
NCHW = (FIELDS * BATCH) // (CHUNK * NW)

_mesh = plsc.VectorSubcoreMesh(core_axis_name="c", subcore_axis_name="s")
_params = pltpu.CompilerParams(use_tc_tiling_on_sc=True,
                               needs_layout_passes=False)


def _wid():
    return lax.axis_index("s") * NC + lax.axis_index("c")


@functools.partial(
    pl.kernel,
    mesh=_mesh,
    out_type=jax.ShapeDtypeStruct((PROWS, 128), jnp.float32),
    scratch_types=[
        pltpu.VMEM((EMBED_DIM, 512), jnp.float32),
        pltpu.VMEM((EMBED_DIM, 512), jnp.float32),
        pltpu.VMEM((128, 128), jnp.float32),
        pltpu.VMEM((128, 128), jnp.float32),
    ]
    + [pltpu.SemaphoreType.DMA] * 4,
    compiler_params=_params,
)
def _pack_kernel(pt_hbm, tail_hbm, packed_hbm,
                 tbuf0, tbuf1, pbuf0, pbuf1,
                 isem0, isem1, osem0, osem1):
    wid = _wid()
    nt = jnp.where(wid < NBLK - (BLK_PER_W - 1) * NW, BLK_PER_W, BLK_PER_W - 1)
    iota16 = lax.iota(jnp.int32, 16)
    tbufs, pbufs = (tbuf0, tbuf1), (pbuf0, pbuf1)
    isem, osem = (isem0, isem1), (osem0, osem1)

    def in_slice(t):
        return pt_hbm.at[:, pl.ds((wid + NW * t) * 512, 512)]

    def out_slice(t):
        return packed_hbm.at[pl.ds((wid + NW * t) * 128, 128)]

    def transpose_block(tb, pb):
        def kk_group(kg, carry):
            vals = []
            for dk in range(4):
                kk = 4 * kg + dk
                col0 = 128 * lax.shift_right_logical(kk, 5) \
                    + 4 * lax.bitwise_and(kk, 31)
                for h in range(8):
                    rows = iota16 + 16 * (h % 2)
                    cols = jnp.full((16,), h // 2, jnp.int32) + col0
                    vals.append(plsc.load_gather(tb, [rows, cols]))
            for dk in range(4):
                kk = 4 * kg + dk
                for h in range(8):
                    pb[kk, pl.ds(16 * h, 16)] = vals[8 * dk + h]
            return carry

        lax.fori_loop(0, 32, kk_group, 0)

    def step(t, s):
        @pl.when(t < nt)
        def _():
            @pl.when(t >= 2)
            def _():
                pltpu.make_async_copy(pbufs[s], out_slice(t), osem[s]).wait()

            @pl.when(t + 1 < nt)
            def _():
                pltpu.async_copy(in_slice(t + 1), tbufs[1 - s], isem[1 - s])

            pltpu.make_async_copy(in_slice(t), tbufs[s], isem[s]).wait()
            transpose_block(tbufs[s], pbufs[s])
            pltpu.async_copy(pbufs[s], out_slice(t), osem[s])

    pltpu.async_copy(in_slice(0), tbuf0, isem0)

    def loop_body(u, carry):
        step(2 * u, 0)
        step(2 * u + 1, 1)
        return carry

    lax.fori_loop(0, (BLK_PER_W + 2) // 2, loop_body, 0)
    for s in range(2):
        pltpu.make_async_copy(pbufs[s], out_slice(0), osem[s]).wait()

    @pl.when(wid == 0)
    def _():
        pltpu.sync_copy(tail_hbm, pbuf0.at[pl.ds(0, TAIL // PACK)])
        pltpu.sync_copy(pbuf0.at[pl.ds(0, TAIL // PACK)],
                        packed_hbm.at[pl.ds(NFC * 32, TAIL // PACK)])


@functools.partial(
    pl.kernel,
    mesh=_mesh,
    out_type=jax.ShapeDtypeStruct((FIELDS, EMBED_DIM, BATCH), jnp.float32),
    scratch_types=[
        pltpu.VMEM((2, 128), jnp.int32),
        pltpu.VMEM((2, 128), jnp.int32),
        pltpu.VMEM((2, 128), jnp.int32),
        pltpu.VMEM((2, 128), jnp.int32),
        pltpu.VMEM((CHUNK, 128), jnp.float32),
        pltpu.VMEM((CHUNK, 128), jnp.float32),
        pltpu.VMEM((EMBED_DIM, CHUNK), jnp.float32),
        pltpu.VMEM((EMBED_DIM, CHUNK), jnp.float32),
    ]
    + [pltpu.SemaphoreType.DMA] * 6,
    compiler_params=_params,
)
def _gather_kernel(q_hbm, o_hbm, packed_hbm, out_hbm,
                   qbuf0, qbuf1, obuf0, obuf1, gbuf0, gbuf1, oblk0, oblk1,
                   ssem0, ssem1, gsem0, gsem1, osem0, osem1):
    wid = _wid()
    iota16 = lax.iota(jnp.int32, 16)
    qbufs, obufs = (qbuf0, qbuf1), (obuf0, obuf1)
    gbufs, oblks = (gbuf0, gbuf1), (oblk0, oblk1)
    ssem, gsem = (ssem0, ssem1), (gsem0, gsem1)
    osem = (osem0, osem1)

    def stage(t, s):
        pltpu.async_copy(q_hbm.at[wid, pl.ds(2 * t, 2)], qbufs[s], ssem[s])
        pltpu.async_copy(o_hbm.at[wid, pl.ds(2 * t, 2)], obufs[s], ssem[s])

    def fire(s):
        pltpu.async_copy(packed_hbm.at[qbufs[s].at[0]],
                         gbufs[s].at[pl.ds(0, 128)], gsem[s])
        pltpu.async_copy(packed_hbm.at[qbufs[s].at[1]],
                         gbufs[s].at[pl.ds(128, 128)], gsem[s])

    def wait_stage(s):
        pltpu.make_async_copy(q_hbm.at[wid, pl.ds(0, 2)], qbufs[s],
                              ssem[s]).wait()
        pltpu.make_async_copy(o_hbm.at[wid, pl.ds(0, 2)], obufs[s],
                              ssem[s]).wait()

    def out_slab(t):
        p = wid * NCHW + t
        f = p // (BATCH // CHUNK)
        bg = lax.rem(p, BATCH // CHUNK)
        return out_hbm.at[f, :, pl.ds(bg * CHUNK, CHUNK)]

    def step(t, s):
        @pl.when(t >= 2)
        def _():
            pltpu.make_async_copy(oblks[s], out_slab(t), osem[s]).wait()

        @pl.when(t + 1 < NCHW)
        def _():
            stage(t + 1, 1 - s)

        pltpu.make_async_copy(packed_hbm.at[pl.ds(0, CHUNK)], gbufs[s],
                              gsem[s]).wait()
        offs = [obufs[s][g // 8, pl.ds(16 * (g % 8), 16)] for g in range(16)]

        def e_group(eg, carry):
            vals = []
            for de in range(2):
                e = 2 * eg + de
                for g in range(16):
                    vals.append(plsc.load_gather(
                        gbufs[s], [iota16 + 16 * g, offs[g] + e]))
            for de in range(2):
                e = 2 * eg + de
                for g in range(16):
                    oblks[s][e, pl.ds(16 * g, 16)] = vals[16 * de + g]
            return carry

        lax.fori_loop(0, 16, e_group, 0)
        pltpu.async_copy(oblks[s], out_slab(t), osem[s])

        @pl.when(t + 1 < NCHW)
        def _():
            wait_stage(1 - s)
            fire(1 - s)

    stage(0, 0)
    wait_stage(0)
    fire(0)

    def loop_body(u, carry):
        step(2 * u, 0)
        step(2 * u + 1, 1)
        return carry

    lax.fori_loop(0, NCHW // 2, loop_body, 0)
    for s in range(2):
        pltpu.make_async_copy(oblks[s], out_slab(0), osem[s]).wait()


def kernel(inputs, params):
    idxt = inputs.astype(jnp.int32).T.reshape(NW, NCHW * 2, 128)
    qarr = jnp.right_shift(idxt, 2)
    oarr = jnp.bitwise_and(idxt, 3) * EMBED_DIM
    tail = params[NFC * 128:].reshape(TAIL // PACK, 128)
    packed = _pack_kernel(params.T, tail)
    out_t = _gather_kernel(qarr, oarr, packed)
    return out_t.transpose(2, 0, 1)


# speedup vs baseline: 1.3874x; 1.0087x over previous
import functools

import jax
import jax.numpy as jnp
from jax import lax
from jax.experimental import pallas as pl
from jax.experimental.pallas import tpu as pltpu
from jax.experimental.pallas import tpu_sc as plsc

NC = 2
NS = 16
NW = NC * NS

BATCH = 16384
FIELDS = 26
EMBED_DIM = 32
VOCAB = 1000000

PACK = 128 // EMBED_DIM
PROWS = VOCAB // PACK
NFC = VOCAB // 128
TAIL = VOCAB - NFC * 128
NBLK = NFC // 4
BLK_PER_W = (NBLK + NW - 1) // NW

CHUNK = 256
NCHW = (FIELDS * BATCH) // (CHUNK * NW)

_mesh = plsc.VectorSubcoreMesh(core_axis_name="c", subcore_axis_name="s")
_params = pltpu.CompilerParams(use_tc_tiling_on_sc=True,
                               needs_layout_passes=False)


def _wid():
    return lax.axis_index("s") * NC + lax.axis_index("c")


@functools.partial(
    pl.kernel,
    mesh=_mesh,
    out_type=jax.ShapeDtypeStruct((PROWS, 128), jnp.float32),
    scratch_types=[
        pltpu.VMEM((EMBED_DIM, 512), jnp.float32),
        pltpu.VMEM((EMBED_DIM, 512), jnp.float32),
        pltpu.VMEM((128, 128), jnp.float32),
        pltpu.VMEM((128, 128), jnp.float32),
    ]
    + [pltpu.SemaphoreType.DMA] * 4,
    compiler_params=_params,
)
def _pack_kernel(pt_hbm, tail_hbm, packed_hbm,
                 tbuf0, tbuf1, pbuf0, pbuf1,
                 isem0, isem1, osem0, osem1):
    wid = _wid()
    nt = jnp.where(wid < NBLK - (BLK_PER_W - 1) * NW, BLK_PER_W, BLK_PER_W - 1)
    iota16 = lax.iota(jnp.int32, 16)
    tbufs, pbufs = (tbuf0, tbuf1), (pbuf0, pbuf1)
    isem, osem = (isem0, isem1), (osem0, osem1)

    def in_slice(t):
        return pt_hbm.at[:, pl.ds((wid + NW * t) * 512, 512)]

    def out_slice(t):
        return packed_hbm.at[pl.ds((wid + NW * t) * 128, 128)]

    def transpose_block(tb, pb):
        def kk_group(kg, carry):
            pairs = []
            for dk in range(4):
                kk = 4 * kg + dk
                col0 = 128 * lax.shift_right_logical(kk, 5) \
                    + 4 * lax.bitwise_and(kk, 31)
                for h in range(8):
                    pairs.append((kk, h, col0))
            vals = [None] * len(pairs)
            lead = 8
            for i in range(len(pairs) + lead):
                if i < len(pairs):
                    kk, h, col0 = pairs[i]
                    rows = iota16 + 16 * (h % 2)
                    cols = jnp.full((16,), h // 2, jnp.int32) + col0
                    vals[i] = plsc.load_gather(tb, [rows, cols])
                if i >= lead:
                    kk, h, _ = pairs[i - lead]
                    pb[kk, pl.ds(16 * h, 16)] = vals[i - lead]
            return carry

        lax.fori_loop(0, 32, kk_group, 0)

    def step(t, s):
        @pl.when(t < nt)
        def _():
            @pl.when(t >= 2)
            def _():
                pltpu.make_async_copy(pbufs[s], out_slice(t), osem[s]).wait()

            @pl.when(t + 1 < nt)
            def _():
                pltpu.async_copy(in_slice(t + 1), tbufs[1 - s], isem[1 - s])

            pltpu.make_async_copy(in_slice(t), tbufs[s], isem[s]).wait()
            transpose_block(tbufs[s], pbufs[s])
            pltpu.async_copy(pbufs[s], out_slice(t), osem[s])

    pltpu.async_copy(in_slice(0), tbuf0, isem0)

    def loop_body(u, carry):
        step(2 * u, 0)
        step(2 * u + 1, 1)
        return carry

    lax.fori_loop(0, (BLK_PER_W + 2) // 2, loop_body, 0)
    for s in range(2):
        pltpu.make_async_copy(pbufs[s], out_slice(0), osem[s]).wait()

    @pl.when(wid == 0)
    def _():
        pltpu.sync_copy(tail_hbm, pbuf0.at[pl.ds(0, TAIL // PACK)])
        pltpu.sync_copy(pbuf0.at[pl.ds(0, TAIL // PACK)],
                        packed_hbm.at[pl.ds(NFC * 32, TAIL // PACK)])


@functools.partial(
    pl.kernel,
    mesh=_mesh,
    out_type=jax.ShapeDtypeStruct((FIELDS, EMBED_DIM, BATCH), jnp.float32),
    scratch_types=[
        pltpu.VMEM((2, 128), jnp.int32),
        pltpu.VMEM((2, 128), jnp.int32),
        pltpu.VMEM((2, 128), jnp.int32),
        pltpu.VMEM((2, 128), jnp.int32),
        pltpu.VMEM((CHUNK, 128), jnp.float32),
        pltpu.VMEM((CHUNK, 128), jnp.float32),
        pltpu.VMEM((EMBED_DIM, CHUNK), jnp.float32),
        pltpu.VMEM((EMBED_DIM, CHUNK), jnp.float32),
    ]
    + [pltpu.SemaphoreType.DMA] * 6,
    compiler_params=_params,
)
def _gather_kernel(q_hbm, o_hbm, packed_hbm, out_hbm,
                   qbuf0, qbuf1, obuf0, obuf1, gbuf0, gbuf1, oblk0, oblk1,
                   ssem0, ssem1, gsem0, gsem1, osem0, osem1):
    wid = _wid()
    iota16 = lax.iota(jnp.int32, 16)
    qbufs, obufs = (qbuf0, qbuf1), (obuf0, obuf1)
    gbufs, oblks = (gbuf0, gbuf1), (oblk0, oblk1)
    ssem, gsem = (ssem0, ssem1), (gsem0, gsem1)
    osem = (osem0, osem1)

    def stage(t, s):
        pltpu.async_copy(q_hbm.at[wid, pl.ds(2 * t, 2)], qbufs[s], ssem[s])
        pltpu.async_copy(o_hbm.at[wid, pl.ds(2 * t, 2)], obufs[s], ssem[s])

    def fire(s):
        pltpu.async_copy(packed_hbm.at[qbufs[s].at[0]],
                         gbufs[s].at[pl.ds(0, 128)], gsem[s])
        pltpu.async_copy(packed_hbm.at[qbufs[s].at[1]],
                         gbufs[s].at[pl.ds(128, 128)], gsem[s])

    def wait_stage(s):
        pltpu.make_async_copy(q_hbm.at[wid, pl.ds(0, 2)], qbufs[s],
                              ssem[s]).wait()
        pltpu.make_async_copy(o_hbm.at[wid, pl.ds(0, 2)], obufs[s],
                              ssem[s]).wait()

    def out_slab(t):
        p = wid * NCHW + t
        f = p // (BATCH // CHUNK)
        bg = lax.rem(p, BATCH // CHUNK)
        return out_hbm.at[f, :, pl.ds(bg * CHUNK, CHUNK)]

    def step(t, s):
        @pl.when(t >= 2)
        def _():
            pltpu.make_async_copy(oblks[s], out_slab(t), osem[s]).wait()

        @pl.when(t + 1 < NCHW)
        def _():
            stage(t + 1, 1 - s)

        pltpu.make_async_copy(packed_hbm.at[pl.ds(0, CHUNK)], gbufs[s],
                              gsem[s]).wait()
        offs = [obufs[s][g // 8, pl.ds(16 * (g % 8), 16)] for g in range(16)]

        def e_group(eg, carry):
            pairs = [(2 * eg + de, g) for de in range(2) for g in range(16)]
            vals = [None] * len(pairs)
            lead = 8
            for i in range(len(pairs) + lead):
                if i < len(pairs):
                    e, g = pairs[i]
                    vals[i] = plsc.load_gather(
                        gbufs[s], [iota16 + 16 * g, offs[g] + e])
                if i >= lead:
                    e, g = pairs[i - lead]
                    oblks[s][e, pl.ds(16 * g, 16)] = vals[i - lead]
            return carry

        lax.fori_loop(0, 16, e_group, 0)
        pltpu.async_copy(oblks[s], out_slab(t), osem[s])

        @pl.when(t + 1 < NCHW)
        def _():
            wait_stage(1 - s)
            fire(1 - s)

    stage(0, 0)
    wait_stage(0)
    fire(0)

    def loop_body(u, carry):
        step(2 * u, 0)
        step(2 * u + 1, 1)
        return carry

    lax.fori_loop(0, NCHW // 2, loop_body, 0)
    for s in range(2):
        pltpu.make_async_copy(oblks[s], out_slab(0), osem[s]).wait()


def kernel(inputs, params):
    idxt = inputs.astype(jnp.int32).T.reshape(NW, NCHW * 2, 128)
    qarr = jnp.right_shift(idxt, 2)
    oarr = jnp.bitwise_and(idxt, 3) * EMBED_DIM
    tail = params[NFC * 128:].reshape(TAIL // PACK, 128)
    packed = _pack_kernel(params.T, tail)
    out_t = _gather_kernel(qarr, oarr, packed)
    return out_t.transpose(2, 0, 1)


# speedup vs baseline: 2.9994x; 2.1619x over previous
import functools

import jax
import jax.numpy as jnp
from jax import lax
from jax.experimental import pallas as pl
from jax.experimental.pallas import tpu as pltpu
from jax.experimental.pallas import tpu_sc as plsc

NC = 2
NS = 16
NW = NC * NS

BATCH = 16384
FIELDS = 26
EMBED_DIM = 32
VOCAB = 1000000

PACK = 128 // EMBED_DIM
PROWS = VOCAB // PACK
NFC = VOCAB // 128
TAIL = VOCAB - NFC * 128
NBLK = NFC // 4
BLK_PER_W = (NBLK + NW - 1) // NW

CHUNK = 128
NCHW = (FIELDS * BATCH) // (CHUNK * NW)

_mesh = plsc.VectorSubcoreMesh(core_axis_name="c", subcore_axis_name="s")
_params = pltpu.CompilerParams(use_tc_tiling_on_sc=True,
                               needs_layout_passes=False)


def _wid():
    return lax.axis_index("s") * NC + lax.axis_index("c")


@functools.partial(
    pl.kernel,
    mesh=_mesh,
    out_type=jax.ShapeDtypeStruct((PROWS, 128), jnp.float32),
    scratch_types=[
        pltpu.VMEM((EMBED_DIM, 512), jnp.float32),
        pltpu.VMEM((EMBED_DIM, 512), jnp.float32),
        pltpu.VMEM((128, 128), jnp.float32),
        pltpu.VMEM((128, 128), jnp.float32),
    ]
    + [pltpu.SemaphoreType.DMA] * 4,
    compiler_params=_params,
)
def _pack_kernel(pt_hbm, tail_hbm, packed_hbm,
                 tbuf0, tbuf1, pbuf0, pbuf1,
                 isem0, isem1, osem0, osem1):
    wid = _wid()
    nt = jnp.where(wid < NBLK - (BLK_PER_W - 1) * NW, BLK_PER_W, BLK_PER_W - 1)
    iota16 = lax.iota(jnp.int32, 16)
    tbufs, pbufs = (tbuf0, tbuf1), (pbuf0, pbuf1)
    isem, osem = (isem0, isem1), (osem0, osem1)

    def in_slice(t):
        return pt_hbm.at[:, pl.ds((wid + NW * t) * 512, 512)]

    def out_slice(t):
        return packed_hbm.at[pl.ds((wid + NW * t) * 128, 128)]

    def transpose_block(tb, pb):
        evecs = (iota16, iota16 + 16)

        def x0_group(xg, carry):
            for dx in range(4):
                x0 = 4 * xg + dx
                for r in range(2):
                    ev = evecs[r]
                    x = lax.bitwise_and(x0 + ev, 511)
                    vals = plsc.load_gather(tb, [ev, x])
                    pr = lax.shift_right_logical(x, 2)
                    pc = lax.shift_left(lax.bitwise_and(x, 3), 5) + ev
                    plsc.store_scatter(pb, [pr, pc], vals)
            return carry

        lax.fori_loop(0, 128, x0_group, 0)

    def step(t, s):
        @pl.when(t < nt)
        def _():
            @pl.when(t >= 2)
            def _():
                pltpu.make_async_copy(pbufs[s], out_slice(t), osem[s]).wait()

            @pl.when(t + 1 < nt)
            def _():
                pltpu.async_copy(in_slice(t + 1), tbufs[1 - s], isem[1 - s])

            pltpu.make_async_copy(in_slice(t), tbufs[s], isem[s]).wait()
            transpose_block(tbufs[s], pbufs[s])
            pltpu.async_copy(pbufs[s], out_slice(t), osem[s])

    pltpu.async_copy(in_slice(0), tbuf0, isem0)

    def loop_body(u, carry):
        step(2 * u, 0)
        step(2 * u + 1, 1)
        return carry

    lax.fori_loop(0, (BLK_PER_W + 2) // 2, loop_body, 0)
    for s in range(2):
        pltpu.make_async_copy(pbufs[s], out_slice(0), osem[s]).wait()

    @pl.when(wid == 0)
    def _():
        pltpu.sync_copy(tail_hbm, pbuf0.at[pl.ds(0, TAIL // PACK)])
        pltpu.sync_copy(pbuf0.at[pl.ds(0, TAIL // PACK)],
                        packed_hbm.at[pl.ds(NFC * 32, TAIL // PACK)])


@functools.partial(
    pl.kernel,
    mesh=_mesh,
    out_type=jax.ShapeDtypeStruct((FIELDS, EMBED_DIM, BATCH), jnp.float32),
    scratch_types=[pltpu.VMEM((1, 128), jnp.int32)] * 8
    + [pltpu.VMEM((CHUNK, 128), jnp.float32)] * 4
    + [pltpu.VMEM((EMBED_DIM, CHUNK), jnp.float32)] * 2
    + [pltpu.SemaphoreType.DMA] * 10,
    compiler_params=_params,
)
def _gather_kernel(q_hbm, o_hbm, packed_hbm, out_hbm, *scr):
    qbufs, obufs = scr[0:4], scr[4:8]
    gbufs, oblks = scr[8:12], scr[12:14]
    ssem, gsem, osem = scr[14:18], scr[18:22], scr[22:24]
    wid = _wid()
    iota16 = lax.iota(jnp.int32, 16)
    bbs = [iota16 + 16 * g for g in range(8)]

    def stage(t, s):
        pltpu.async_copy(q_hbm.at[wid, pl.ds(t, 1)], qbufs[s], ssem[s])
        pltpu.async_copy(o_hbm.at[wid, pl.ds(t, 1)], obufs[s], ssem[s])

    def wait_stage(s):
        pltpu.make_async_copy(q_hbm.at[wid, pl.ds(0, 1)], qbufs[s],
                              ssem[s]).wait()
        pltpu.make_async_copy(o_hbm.at[wid, pl.ds(0, 1)], obufs[s],
                              ssem[s]).wait()

    def fire(s):
        pltpu.async_copy(packed_hbm.at[qbufs[s].at[0]], gbufs[s], gsem[s])

    def out_slab(t):
        p = wid * NCHW + t
        f = p // (BATCH // CHUNK)
        bg = lax.rem(p, BATCH // CHUNK)
        return out_hbm.at[f, :, pl.ds(bg * CHUNK, CHUNK)]

    def step(t, s4, s2):
        @pl.when(t + 3 < NCHW)
        def _():
            stage(t + 3, (s4 + 3) % 4)

        @pl.when(t + 2 < NCHW)
        def _():
            wait_stage((s4 + 2) % 4)
            fire((s4 + 2) % 4)

        @pl.when(t >= 2)
        def _():
            pltpu.make_async_copy(oblks[s2], out_slab(t), osem[s2]).wait()

        pltpu.make_async_copy(packed_hbm.at[pl.ds(0, CHUNK)], gbufs[s4],
                              gsem[s4]).wait()
        offs = [obufs[s4][0, pl.ds(16 * g, 16)] for g in range(8)]

        def e_group(e0, carry):
            ev = lax.bitwise_and(e0 + iota16, 31)
            for g in range(8):
                vals = plsc.load_gather(gbufs[s4], [bbs[g], offs[g] + ev])
                plsc.store_scatter(oblks[s2], [ev, bbs[g]], vals)
            return carry

        lax.fori_loop(0, 32, e_group, 0)
        pltpu.async_copy(oblks[s2], out_slab(t), osem[s2])

    stage(0, 0)
    stage(1, 1)
    stage(2, 2)
    wait_stage(0)
    fire(0)
    wait_stage(1)
    fire(1)

    def loop_body(u, carry):
        for k in range(4):
            step(4 * u + k, k, k % 2)
        return carry

    lax.fori_loop(0, NCHW // 4, loop_body, 0)
    for s in range(2):
        pltpu.make_async_copy(oblks[s], out_slab(0), osem[s]).wait()


def kernel(inputs, params):
    idxt = inputs.astype(jnp.int32).T.reshape(NW, NCHW, 128)
    qarr = jnp.right_shift(idxt, 2)
    oarr = jnp.bitwise_and(idxt, 3) * EMBED_DIM
    tail = params[NFC * 128:].reshape(TAIL // PACK, 128)
    packed = _pack_kernel(params.T, tail)
    out_t = _gather_kernel(qarr, oarr, packed)
    return out_t.transpose(2, 0, 1)
